# baseline (speedup 1.0000x reference)
import functools

import jax
import jax.numpy as jnp
from jax import lax
from jax.experimental import pallas as pl
from jax.experimental.pallas import tpu as pltpu
from jax.experimental.pallas import tpu_sc as plsc

_P = 0.02
_NC = 2
_NS = 16
_NW = _NC * _NS
_CHUNK = 2048


def _gather_sc(table, idx_flat):
    n = idx_flat.shape[0]
    per_w = n // _NW
    mesh = plsc.VectorSubcoreMesh(core_axis_name="c", subcore_axis_name="s")

    @functools.partial(
        pl.kernel,
        out_type=jax.ShapeDtypeStruct((n,), jnp.float32),
        mesh=mesh,
        scratch_types=[
            pltpu.VMEM((_CHUNK,), jnp.int32),
            pltpu.VMEM((_CHUNK,), jnp.float32),
            pltpu.SemaphoreType.DMA,
        ],
    )
    def gather_kernel(table_hbm, idx_hbm, out_hbm, idx_v, val_v, sem):
        wid = lax.axis_index("s") * _NC + lax.axis_index("c")
        base = wid * per_w

        @pl.loop(0, per_w, step=_CHUNK)
        def _(off):
            pltpu.sync_copy(idx_hbm.at[pl.ds(base + off, _CHUNK)], idx_v)
            pltpu.async_copy(table_hbm.at[idx_v], val_v, sem).wait()
            pltpu.sync_copy(val_v, out_hbm.at[pl.ds(base + off, _CHUNK)])

    return gather_kernel(table, idx_flat)


def _soften_tc(table):
    v = table.shape[0]

    def body(w_ref, o_ref):
        o_ref[...] = jnp.exp(_P * jnp.log(w_ref[...]))

    t2d = table.reshape(1000, v // 1000)
    out = pl.pallas_call(
        body,
        out_shape=jax.ShapeDtypeStruct(t2d.shape, jnp.float32),
    )(t2d)
    return out.reshape(v)


def _finish_tc(w, t):
    b, s = w.shape
    blk = 1024

    def body(w_ref, t_ref, o_ref):
        soft = w_ref[...]
        mu = jnp.mean(soft, axis=1, keepdims=True)
        tv = t_ref[...]
        down = soft * (tv / (mu + 1e-10))
        up = 1.0 - (1.0 - soft) * ((1.0 - tv) / (1.0 - mu + 1e-10))
        o_ref[...] = jnp.clip(jnp.where(mu > tv, down, up), 0.0, 1.0)

    return pl.pallas_call(
        body,
        grid=(b // blk,),
        in_specs=[
            pl.BlockSpec((blk, s), lambda i: (i, 0)),
            pl.BlockSpec((blk, 1), lambda i: (i, 0)),
        ],
        out_specs=pl.BlockSpec((blk, s), lambda i: (i, 0)),
        out_shape=jax.ShapeDtypeStruct((b, s), jnp.float32),
    )(w, t)


def kernel(base_weights, x, target_mask_rate):
    b, s = x.shape
    softened_table = _soften_tc(base_weights)
    soft_flat = _gather_sc(softened_table, x.reshape(-1))
    return _finish_tc(soft_flat.reshape(b, s), target_mask_rate)


# speedup vs baseline: 1.5169x; 1.5169x over previous
import functools

import jax
import jax.numpy as jnp
from jax import lax
from jax.experimental import pallas as pl
from jax.experimental.pallas import tpu as pltpu
from jax.experimental.pallas import tpu_sc as plsc

_P = 0.02
_NC = 2
_NS = 16
_NW = _NC * _NS
_CHUNK = 2048


def _gather_sc(table, idx_flat):
    n = idx_flat.shape[0]
    per_w = n // _NW
    mesh = plsc.VectorSubcoreMesh(core_axis_name="c", subcore_axis_name="s")

    v = table.shape[0]
    stage = 62496

    @functools.partial(
        pl.kernel,
        out_type=jax.ShapeDtypeStruct((n,), jnp.float32),
        mesh=mesh,
        scratch_types=[
            pltpu.VMEM((_CHUNK,), jnp.int32),
            pltpu.VMEM((_CHUNK,), jnp.float32),
            pltpu.VMEM_SHARED((v,), jnp.float32),
            pltpu.VMEM((stage,), jnp.float32),
            pltpu.SemaphoreType.DMA,
        ],
    )
    def gather_kernel(table_hbm, idx_hbm, out_hbm, idx_v, val_v, table_sp,
                      stage_v, sem):
        sid = lax.axis_index("s")
        wid = sid * _NC + lax.axis_index("c")
        base = wid * per_w

        off0 = sid * stage
        pltpu.sync_copy(table_hbm.at[pl.ds(off0, stage)], stage_v)
        pltpu.sync_copy(stage_v, table_sp.at[pl.ds(off0, stage)])
        tail = v - _NS * stage
        if tail:
            @pl.when(sid == _NS - 1)
            def _():
                pltpu.sync_copy(table_hbm.at[pl.ds(_NS * stage, tail)],
                                stage_v.at[pl.ds(0, tail)])
                pltpu.sync_copy(stage_v.at[pl.ds(0, tail)],
                                table_sp.at[pl.ds(_NS * stage, tail)])
        plsc.subcore_barrier()

        @pl.loop(0, per_w, step=_CHUNK)
        def _(off):
            pltpu.sync_copy(idx_hbm.at[pl.ds(base + off, _CHUNK)], idx_v)
            pltpu.async_copy(table_sp.at[idx_v], val_v, sem).wait()
            pltpu.sync_copy(val_v, out_hbm.at[pl.ds(base + off, _CHUNK)])

    return gather_kernel(table, idx_flat)


def _soften_tc(table):
    v = table.shape[0]

    def body(w_ref, o_ref):
        o_ref[...] = jnp.exp(_P * jnp.log(w_ref[...]))

    t2d = table.reshape(1000, v // 1000)
    out = pl.pallas_call(
        body,
        out_shape=jax.ShapeDtypeStruct(t2d.shape, jnp.float32),
    )(t2d)
    return out.reshape(v)


def _finish_tc(w, t):
    b, s = w.shape
    blk = 1024

    def body(w_ref, t_ref, o_ref):
        soft = w_ref[...]
        mu = jnp.mean(soft, axis=1, keepdims=True)
        tv = t_ref[...]
        down = soft * (tv / (mu + 1e-10))
        up = 1.0 - (1.0 - soft) * ((1.0 - tv) / (1.0 - mu + 1e-10))
        o_ref[...] = jnp.clip(jnp.where(mu > tv, down, up), 0.0, 1.0)

    return pl.pallas_call(
        body,
        grid=(b // blk,),
        in_specs=[
            pl.BlockSpec((blk, s), lambda i: (i, 0)),
            pl.BlockSpec((blk, 1), lambda i: (i, 0)),
        ],
        out_specs=pl.BlockSpec((blk, s), lambda i: (i, 0)),
        out_shape=jax.ShapeDtypeStruct((b, s), jnp.float32),
    )(w, t)


def kernel(base_weights, x, target_mask_rate):
    b, s = x.shape
    softened_table = _soften_tc(base_weights)
    soft_flat = _gather_sc(softened_table, x.reshape(-1))
    return _finish_tc(soft_flat.reshape(b, s), target_mask_rate)


# speedup vs baseline: 1.8565x; 1.2239x over previous
import functools

import jax
import jax.numpy as jnp
from jax import lax
from jax.experimental import pallas as pl
from jax.experimental.pallas import tpu as pltpu
from jax.experimental.pallas import tpu_sc as plsc

_P = 0.02
_NC = 2
_NS = 16
_NW = _NC * _NS
_CHUNK = 12800


def _gather_sc(table, idx_flat):
    n = idx_flat.shape[0]
    per_w = n // _NW
    mesh = plsc.VectorSubcoreMesh(core_axis_name="c", subcore_axis_name="s")

    v = table.shape[0]
    stage = 10000
    n_slices = v // stage
    assert n_slices * stage == v
    n_rounds = -(-n_slices // _NS)

    n_win = per_w // _CHUNK
    assert n_win % 2 == 0 and n_win * _CHUNK == per_w

    @functools.partial(
        pl.kernel,
        out_type=jax.ShapeDtypeStruct((n,), jnp.float32),
        mesh=mesh,
        scratch_types=[
            pltpu.VMEM((_CHUNK,), jnp.int32),
            pltpu.VMEM((_CHUNK,), jnp.int32),
            pltpu.VMEM((_CHUNK,), jnp.float32),
            pltpu.VMEM((_CHUNK,), jnp.float32),
            pltpu.VMEM_SHARED((v,), jnp.float32),
            pltpu.SemaphoreType.DMA,
            pltpu.SemaphoreType.DMA,
            pltpu.SemaphoreType.DMA,
            pltpu.SemaphoreType.DMA,
            pltpu.SemaphoreType.DMA,
        ],
    )
    def gather_kernel(table_hbm, idx_hbm, out_hbm, ib0, ib1, vb0, vb1,
                      table_sp, ia0, ia1, oa0, oa1, gs):
        sid = lax.axis_index("s")
        wid = sid * _NC + lax.axis_index("c")
        base = wid * per_w

        for r in range(n_rounds):
            slice_id = r * _NS + sid

            @pl.when(slice_id < n_slices)
            def _():
                so = slice_id * stage
                pltpu.sync_copy(table_hbm.at[pl.ds(so, stage)],
                                vb0.at[pl.ds(0, stage)])
                pltpu.sync_copy(vb0.at[pl.ds(0, stage)],
                                table_sp.at[pl.ds(so, stage)])

        plsc.subcore_barrier()

        pltpu.async_copy(idx_hbm.at[pl.ds(base, _CHUNK)], ib0, ia0)
        pltpu.async_copy(idx_hbm.at[pl.ds(base + _CHUNK, _CHUNK)], ib1, ia1)

        @pl.loop(0, n_win, step=2)
        def _(w):
            for k, (ib, vb, ia, oa) in enumerate(
                    ((ib0, vb0, ia0, oa0), (ib1, vb1, ia1, oa1))):
                off = base + (w + k) * _CHUNK
                pltpu.make_async_copy(idx_hbm.at[pl.ds(0, _CHUNK)], ib, ia).wait()

                @pl.when(w + k >= 2)
                def _():
                    pltpu.make_async_copy(
                        out_hbm.at[pl.ds(0, _CHUNK)], vb, oa).wait()

                pltpu.async_copy(table_sp.at[ib], vb, gs).wait()
                pltpu.async_copy(vb, out_hbm.at[pl.ds(off, _CHUNK)], oa)

                @pl.when(w + k + 2 < n_win)
                def _():
                    pltpu.async_copy(
                        idx_hbm.at[pl.ds(off + 2 * _CHUNK, _CHUNK)], ib, ia)

        pltpu.make_async_copy(out_hbm.at[pl.ds(0, _CHUNK)], vb0, oa0).wait()
        pltpu.make_async_copy(out_hbm.at[pl.ds(0, _CHUNK)], vb1, oa1).wait()

    return gather_kernel(table, idx_flat)


def _soften_tc(table):
    v = table.shape[0]

    def body(w_ref, o_ref):
        o_ref[...] = jnp.exp(_P * jnp.log(w_ref[...]))

    t2d = table.reshape(1000, v // 1000)
    out = pl.pallas_call(
        body,
        out_shape=jax.ShapeDtypeStruct(t2d.shape, jnp.float32),
    )(t2d)
    return out.reshape(v)


def _finish_tc(w, t):
    b, s = w.shape
    blk = 1024

    def body(w_ref, t_ref, o_ref):
        soft = w_ref[...]
        mu = jnp.mean(soft, axis=1, keepdims=True)
        tv = t_ref[...]
        down = soft * (tv / (mu + 1e-10))
        up = 1.0 - (1.0 - soft) * ((1.0 - tv) / (1.0 - mu + 1e-10))
        o_ref[...] = jnp.clip(jnp.where(mu > tv, down, up), 0.0, 1.0)

    return pl.pallas_call(
        body,
        grid=(b // blk,),
        in_specs=[
            pl.BlockSpec((blk, s), lambda i: (i, 0)),
            pl.BlockSpec((blk, 1), lambda i: (i, 0)),
        ],
        out_specs=pl.BlockSpec((blk, s), lambda i: (i, 0)),
        out_shape=jax.ShapeDtypeStruct((b, s), jnp.float32),
    )(w, t)


def kernel(base_weights, x, target_mask_rate):
    b, s = x.shape
    softened_table = _soften_tc(base_weights)
    soft_flat = _gather_sc(softened_table, x.reshape(-1))
    return _finish_tc(soft_flat.reshape(b, s), target_mask_rate)
